# Initial kernel scaffold; baseline (speedup 1.0000x reference)
import jax
import jax.numpy as jnp
from jax.experimental import pallas as pl


def kernel(inputs, weights_map):
    raise NotImplementedError("write your pallas kernel here")


# speedup vs baseline: 2.3174x; 2.3174x over previous
import functools

import jax
import jax.numpy as jnp
from jax.experimental import pallas as pl

GRID_H, GRID_W, INPUT_DIM = 32, 32, 256
K = GRID_H * GRID_W
BLOCK_B = 512


def _bmu_block_kernel(x_ref, w_ref, idx_ref, qerr_ref):
    x = x_ref[...]
    w = w_ref[...]
    x_sq = jnp.sum(x * x, axis=1, keepdims=True)
    w_sq = jnp.sum(w * w, axis=1)[None, :]
    cross = jax.lax.dot_general(
        x, w,
        dimension_numbers=(((1,), (1,)), ((), ())),
        preferred_element_type=jnp.float32,
    )
    dist = jnp.maximum(x_sq - 2.0 * cross + w_sq, 0.0)
    minval = jnp.min(dist, axis=1, keepdims=True)
    lane = jax.lax.broadcasted_iota(jnp.int32, dist.shape, 1)
    idx = jnp.min(jnp.where(dist == minval, lane, K), axis=1)
    idx_ref[...] = idx
    qerr_ref[...] = jnp.sqrt(minval[:, 0])


@functools.partial(jax.jit)
def _bmu_search(inputs, flat_weights):
    batch = inputs.shape[0]
    grid = (batch // BLOCK_B,)
    return pl.pallas_call(
        _bmu_block_kernel,
        grid=grid,
        in_specs=[
            pl.BlockSpec((BLOCK_B, INPUT_DIM), lambda i: (i, 0)),
            pl.BlockSpec((K, INPUT_DIM), lambda i: (0, 0)),
        ],
        out_specs=[
            pl.BlockSpec((BLOCK_B,), lambda i: (i,)),
            pl.BlockSpec((BLOCK_B,), lambda i: (i,)),
        ],
        out_shape=[
            jax.ShapeDtypeStruct((batch,), jnp.int32),
            jax.ShapeDtypeStruct((batch,), jnp.float32),
        ],
    )(inputs, flat_weights)


def kernel(inputs, weights_map):
    flat_weights = jnp.reshape(weights_map, (K, INPUT_DIM))
    idx, qerr = _bmu_search(inputs, flat_weights)
    bmu_y = idx // GRID_W
    bmu_x = idx % GRID_W
    bmu_indices = jnp.stack([bmu_y, bmu_x], axis=1)
    return bmu_indices, qerr


# speedup vs baseline: 3.1648x; 1.3656x over previous
import functools

import jax
import jax.numpy as jnp
from jax.experimental import pallas as pl

GRID_H, GRID_W, INPUT_DIM = 32, 32, 256
K = GRID_H * GRID_W
BLOCK_B = 512


def _bmu_block_kernel(x_ref, w_ref, idx_ref, qerr_ref):
    x = x_ref[...]
    w = w_ref[...]
    ones_d = jnp.ones((1, INPUT_DIM), jnp.float32)
    x_sq = jax.lax.dot_general(
        ones_d, x * x,
        dimension_numbers=(((1,), (1,)), ((), ())),
        preferred_element_type=jnp.float32,
    )
    w_sq = jnp.sum(w * w, axis=1, keepdims=True)
    cross = jax.lax.dot_general(
        w, x,
        dimension_numbers=(((1,), (1,)), ((), ())),
        preferred_element_type=jnp.float32,
    )
    dist = jnp.maximum((x_sq - 2.0 * cross) + w_sq, 0.0)
    minval = jnp.min(dist, axis=0, keepdims=True)
    row = jax.lax.broadcasted_iota(jnp.int32, dist.shape, 0)
    idx = jnp.min(jnp.where(dist == minval, row, K), axis=0)
    idx_ref[...] = idx
    qerr_ref[...] = jnp.sqrt(minval[0])


@functools.partial(jax.jit)
def _bmu_search(inputs, flat_weights):
    batch = inputs.shape[0]
    grid = (batch // BLOCK_B,)
    return pl.pallas_call(
        _bmu_block_kernel,
        grid=grid,
        in_specs=[
            pl.BlockSpec((BLOCK_B, INPUT_DIM), lambda i: (i, 0)),
            pl.BlockSpec((K, INPUT_DIM), lambda i: (0, 0)),
        ],
        out_specs=[
            pl.BlockSpec((BLOCK_B,), lambda i: (i,)),
            pl.BlockSpec((BLOCK_B,), lambda i: (i,)),
        ],
        out_shape=[
            jax.ShapeDtypeStruct((batch,), jnp.int32),
            jax.ShapeDtypeStruct((batch,), jnp.float32),
        ],
    )(inputs, flat_weights)


def kernel(inputs, weights_map):
    flat_weights = jnp.reshape(weights_map, (K, INPUT_DIM))
    idx, qerr = _bmu_search(inputs, flat_weights)
    bmu_y = idx // GRID_W
    bmu_x = idx % GRID_W
    bmu_indices = jnp.stack([bmu_y, bmu_x], axis=1)
    return bmu_indices, qerr


# speedup vs baseline: 4.4195x; 1.3965x over previous
import functools

import jax
import jax.numpy as jnp
from jax.experimental import pallas as pl
from jax.experimental.pallas import tpu as pltpu

GRID_H, GRID_W, INPUT_DIM = 32, 32, 256
K = GRID_H * GRID_W
BLOCK_B = 1024
SUB = 8


def _bmu_block_kernel(x_ref, w_ref, idx_ref, qerr_ref, wsq_ref):
    x = x_ref[...]
    w = w_ref[...]

    @pl.when(pl.program_id(0) == 0)
    def _():
        wsq_ref[...] = jnp.sum(w * w, axis=1, keepdims=True)

    ones_d = jnp.ones((1, INPUT_DIM), jnp.float32)
    x_sq = jax.lax.dot_general(
        ones_d, x * x,
        dimension_numbers=(((1,), (1,)), ((), ())),
        preferred_element_type=jnp.float32,
    )
    cross = jax.lax.dot_general(
        w, -2.0 * x,
        dimension_numbers=(((1,), (1,)), ((), ())),
        preferred_element_type=jnp.float32,
    )
    wsq = wsq_ref[...]

    best = jnp.full((SUB, BLOCK_B), jnp.inf, jnp.float32)
    bestrow = jnp.zeros((SUB, BLOCK_B), jnp.int32)
    for r in range(K // SUB):
        d = jnp.maximum((x_sq + cross[r * SUB:(r + 1) * SUB]) +
                        wsq[r * SUB:(r + 1) * SUB], 0.0)
        m = d < best
        best = jnp.where(m, d, best)
        bestrow = jnp.where(m, r, bestrow)

    k = bestrow * SUB + jax.lax.broadcasted_iota(jnp.int32, best.shape, 0)
    minv = jnp.min(best, axis=0, keepdims=True)
    idx = jnp.min(jnp.where(best == minv, k, K), axis=0)
    idx_ref[...] = idx
    qerr_ref[...] = jnp.sqrt(minv[0])


@functools.partial(jax.jit)
def _bmu_search(inputs, flat_weights):
    batch = inputs.shape[0]
    grid = (batch // BLOCK_B,)
    return pl.pallas_call(
        _bmu_block_kernel,
        grid=grid,
        in_specs=[
            pl.BlockSpec((BLOCK_B, INPUT_DIM), lambda i: (i, 0)),
            pl.BlockSpec((K, INPUT_DIM), lambda i: (0, 0)),
        ],
        out_specs=[
            pl.BlockSpec((BLOCK_B,), lambda i: (i,)),
            pl.BlockSpec((BLOCK_B,), lambda i: (i,)),
        ],
        out_shape=[
            jax.ShapeDtypeStruct((batch,), jnp.int32),
            jax.ShapeDtypeStruct((batch,), jnp.float32),
        ],
        scratch_shapes=[pltpu.VMEM((K, 1), jnp.float32)],
    )(inputs, flat_weights)


def kernel(inputs, weights_map):
    flat_weights = jnp.reshape(weights_map, (K, INPUT_DIM))
    idx, qerr = _bmu_search(inputs, flat_weights)
    bmu_y = idx // GRID_W
    bmu_x = idx % GRID_W
    bmu_indices = jnp.stack([bmu_y, bmu_x], axis=1)
    return bmu_indices, qerr


# speedup vs baseline: 4.5923x; 1.0391x over previous
import functools

import jax
import jax.numpy as jnp
from jax.experimental import pallas as pl
from jax.experimental.pallas import tpu as pltpu

GRID_H, GRID_W, INPUT_DIM = 32, 32, 256
K = GRID_H * GRID_W
BLOCK_B = 4096
SLAB_K = 256
SUB = 8


def _bmu_block_kernel(x_ref, w_ref, idx_ref, qerr_ref, wsq_ref):
    x = x_ref[...]
    w = w_ref[...]

    @pl.when(pl.program_id(0) == 0)
    def _():
        wsq_ref[...] = jnp.sum(w * w, axis=1, keepdims=True)

    ones_d = jnp.ones((1, INPUT_DIM), jnp.float32)
    x_sq = jax.lax.dot_general(
        ones_d, x * x,
        dimension_numbers=(((1,), (1,)), ((), ())),
        preferred_element_type=jnp.float32,
    )
    m2x = -2.0 * x
    wsq = wsq_ref[...]

    best = jnp.full((SUB, BLOCK_B), jnp.inf, jnp.float32)
    bestrow = jnp.zeros((SUB, BLOCK_B), jnp.int32)
    for s in range(K // SLAB_K):
        cross = jax.lax.dot_general(
            w[s * SLAB_K:(s + 1) * SLAB_K], m2x,
            dimension_numbers=(((1,), (1,)), ((), ())),
            preferred_element_type=jnp.float32,
        )
        for c in range(SLAB_K // SUB):
            r = s * (SLAB_K // SUB) + c
            d = jnp.maximum((x_sq + cross[c * SUB:(c + 1) * SUB]) +
                            wsq[r * SUB:(r + 1) * SUB], 0.0)
            m = d < best
            best = jnp.where(m, d, best)
            bestrow = jnp.where(m, r, bestrow)

    k = bestrow * SUB + jax.lax.broadcasted_iota(jnp.int32, best.shape, 0)
    minv = jnp.min(best, axis=0, keepdims=True)
    idx = jnp.min(jnp.where(best == minv, k, K), axis=0)
    idx_ref[...] = idx
    qerr_ref[...] = jnp.sqrt(minv[0])


@functools.partial(jax.jit)
def _bmu_search(inputs, flat_weights):
    batch = inputs.shape[0]
    grid = (batch // BLOCK_B,)
    return pl.pallas_call(
        _bmu_block_kernel,
        grid=grid,
        in_specs=[
            pl.BlockSpec((BLOCK_B, INPUT_DIM), lambda i: (i, 0)),
            pl.BlockSpec((K, INPUT_DIM), lambda i: (0, 0)),
        ],
        out_specs=[
            pl.BlockSpec((BLOCK_B,), lambda i: (i,)),
            pl.BlockSpec((BLOCK_B,), lambda i: (i,)),
        ],
        out_shape=[
            jax.ShapeDtypeStruct((batch,), jnp.int32),
            jax.ShapeDtypeStruct((batch,), jnp.float32),
        ],
        scratch_shapes=[pltpu.VMEM((K, 1), jnp.float32)],
    )(inputs, flat_weights)


def kernel(inputs, weights_map):
    flat_weights = jnp.reshape(weights_map, (K, INPUT_DIM))
    idx, qerr = _bmu_search(inputs, flat_weights)
    bmu_y = idx // GRID_W
    bmu_x = idx % GRID_W
    bmu_indices = jnp.stack([bmu_y, bmu_x], axis=1)
    return bmu_indices, qerr
